# baseline (speedup 1.0000x reference)
import functools

import jax
import jax.numpy as jnp
import numpy as np
from jax import lax
from jax.experimental import pallas as pl
from jax.experimental.pallas import tpu as pltpu
from jax.experimental.pallas import tpu_sc as plsc

B, N, M, CIN, C1, C2 = 4, 16384, 1024, 32, 64, 64
BLK = 512

_NC, _NS = 2, 16
_NW = _NC * _NS
_FLAT = 3 * B * N
_PER_W = _FLAT // _NW
_CHUNK = 2048
_NT = _PER_W // _CHUNK


def _knn_body(p_ref, c_ref, i_ref, w_ref):
    b = pl.program_id(0)
    p = p_ref[0]
    c = c_ref[0]
    pn2 = jnp.sum(p * p, axis=0)
    cm2 = jnp.sum(c * c, axis=0)
    cp = lax.dot_general(c, p, (((0,), (0,)), ((), ())),
                         preferred_element_type=jnp.float32)
    d2 = cm2[:, None] - 2.0 * cp + pn2[None, :]

    iota = lax.broadcasted_iota(jnp.int32, d2.shape, 0)
    inf = jnp.float32(np.inf)
    vals, idxs = [], []
    dcur = d2
    for _ in range(3):
        v = jnp.min(dcur, axis=0)
        i = jnp.min(jnp.where(dcur == v[None, :], iota, M), axis=0)
        vals.append(v)
        idxs.append(i)
        dcur = jnp.where(iota == i[None, :], inf, dcur)

    w = [1.0 / (jnp.sqrt(jnp.maximum(v, 1e-10)) + 1e-8) for v in vals]
    wsum = w[0] + w[1] + w[2]
    i_ref[0] = jnp.concatenate([(i + b * M)[None] for i in idxs], axis=0)
    w_ref[0] = jnp.concatenate([(wi / wsum)[None] for wi in w], axis=0)


def _knn_call(points_coords, centers_coords):
    return pl.pallas_call(
        _knn_body,
        grid=(B, N // BLK),
        in_specs=[
            pl.BlockSpec((1, 3, BLK), lambda b, j: (b, 0, j)),
            pl.BlockSpec((1, 3, M), lambda b, j: (b, 0, 0)),
        ],
        out_specs=[
            pl.BlockSpec((1, 3, BLK), lambda b, j: (b, 0, j)),
            pl.BlockSpec((1, 3, BLK), lambda b, j: (b, 0, j)),
        ],
        out_shape=[
            jax.ShapeDtypeStruct((B, 3, N), jnp.int32),
            jax.ShapeDtypeStruct((B, 3, N), jnp.float32),
        ],
    )(points_coords, centers_coords)


_GB = 128
_NG = _CHUNK // _GB


def _sc_gather_body(table_hbm, idx_hbm, out_hbm, idx_v, rows_v, sem):
    wid = lax.axis_index("s") * _NC + lax.axis_index("c")

    def chunk(t, _):
        base = wid * _PER_W + t * _CHUNK
        pltpu.sync_copy(idx_hbm.at[pl.ds(base // _GB, _NG)], idx_v)
        copies = [
            pltpu.async_copy(table_hbm.at[idx_v.at[j]],
                             rows_v.at[pl.ds(j * _GB, _GB)], sem)
            for j in range(_NG)
        ]
        for c in copies:
            c.wait()
        pltpu.sync_copy(rows_v, out_hbm.at[pl.ds(base, _CHUNK)])
        return ()

    lax.fori_loop(0, _NT, chunk, ())


@functools.cache
def _sc_gather():
    return pl.kernel(
        _sc_gather_body,
        out_type=jax.ShapeDtypeStruct((_FLAT, CIN), jnp.float32),
        mesh=plsc.VectorSubcoreMesh(core_axis_name="c", subcore_axis_name="s",
                                    num_cores=_NC, num_subcores=_NS),
        scratch_types=[
            pltpu.VMEM((_NG, _GB), jnp.int32),
            pltpu.VMEM((_CHUNK, CIN), jnp.float32),
            pltpu.SemaphoreType.DMA,
        ],
        compiler_params=pltpu.CompilerParams(use_tc_tiling_on_sc=False),
    )


def _mlp_body(g_ref, w_ref, w1_ref, b1_ref, w2_ref, b2_ref, o_ref):
    g = g_ref[0]
    w = w_ref[0]
    x = (g[0] * w[0][:, None] + g[1] * w[1][:, None] + g[2] * w[2][:, None])
    xt = jnp.swapaxes(x, 0, 1)
    h1 = jnp.maximum(
        lax.dot_general(w1_ref[...], xt, (((1,), (0,)), ((), ())),
                        preferred_element_type=jnp.float32) + b1_ref[...], 0.0)
    h2 = jnp.maximum(
        lax.dot_general(w2_ref[...], h1, (((1,), (0,)), ((), ())),
                        preferred_element_type=jnp.float32) + b2_ref[...], 0.0)
    o_ref[0] = h2


def _mlp_call(gath, w, W1, b1, W2, b2):
    return pl.pallas_call(
        _mlp_body,
        grid=(B, N // BLK),
        in_specs=[
            pl.BlockSpec((1, 3, BLK, CIN), lambda b, j: (b, 0, j, 0)),
            pl.BlockSpec((1, 3, BLK), lambda b, j: (b, 0, j)),
            pl.BlockSpec((C1, CIN), lambda b, j: (0, 0)),
            pl.BlockSpec((C1, 1), lambda b, j: (0, 0)),
            pl.BlockSpec((C2, C1), lambda b, j: (0, 0)),
            pl.BlockSpec((C2, 1), lambda b, j: (0, 0)),
        ],
        out_specs=pl.BlockSpec((1, C2, BLK), lambda b, j: (b, 0, j)),
        out_shape=jax.ShapeDtypeStruct((B, C2, N), jnp.float32),
    )(gath, w, W1, b1.reshape(C1, 1), W2, b2.reshape(C2, 1))


def kernel(points_coords, centers_coords, centers_features, condition, W1, b1, W2, b2):
    idx, w = _knn_call(points_coords, centers_coords)
    table = jnp.swapaxes(centers_features, 1, 2).reshape(B * M, CIN)
    gath = _sc_gather()(table, idx.reshape(_FLAT // _GB, _GB))
    x = _mlp_call(gath.reshape(B, 3, N, CIN), w, W1, b1, W2, b2)
    return (x, points_coords, condition)


# speedup vs baseline: 1.4837x; 1.4837x over previous
import functools

import jax
import jax.numpy as jnp
import numpy as np
from jax import lax
from jax.experimental import pallas as pl
from jax.experimental.pallas import tpu as pltpu
from jax.experimental.pallas import tpu_sc as plsc

B, N, M, CIN, C1, C2 = 4, 16384, 1024, 32, 64, 64
BLK = 512

_NC, _NS = 2, 16
_NW = _NC * _NS
_FLAT = 3 * B * N
_PER_W = _FLAT // _NW
_CHUNK = 2048
_NT = _PER_W // _CHUNK


def _knn_body(p_ref, c_ref, i_ref, w_ref):
    b = pl.program_id(0)
    p = p_ref[0]
    c = c_ref[0]
    pn2 = jnp.sum(p * p, axis=0)
    cm2 = jnp.sum(c * c, axis=0)
    cp = lax.dot_general(c, p, (((0,), (0,)), ((), ())),
                         preferred_element_type=jnp.float32)
    d2 = cm2[:, None] - 2.0 * cp + pn2[None, :]

    iota = lax.broadcasted_iota(jnp.int32, d2.shape, 0)
    d2i = lax.bitcast_convert_type(d2, jnp.int32)
    key = lax.bitcast_convert_type((d2i & ~(M - 1)) | iota, jnp.float32)

    inf = jnp.float32(np.inf)
    vals, idxs = [], []
    for k in range(3):
        v = jnp.min(key, axis=0)
        vi = lax.bitcast_convert_type(v, jnp.int32)
        idxs.append(vi & (M - 1))
        vals.append(lax.bitcast_convert_type(vi & ~(M - 1), jnp.float32))
        if k < 2:
            key = jnp.where(key == v[None, :], inf, key)

    w = [1.0 / (jnp.sqrt(jnp.maximum(v, 1e-10)) + 1e-8) for v in vals]
    wsum = w[0] + w[1] + w[2]
    i_ref[0] = jnp.concatenate([(i + b * M)[None] for i in idxs], axis=0)
    w_ref[0] = jnp.concatenate([(wi / wsum)[None] for wi in w], axis=0)


def _knn_call(points_coords, centers_coords):
    return pl.pallas_call(
        _knn_body,
        grid=(B, N // BLK),
        in_specs=[
            pl.BlockSpec((1, 3, BLK), lambda b, j: (b, 0, j)),
            pl.BlockSpec((1, 3, M), lambda b, j: (b, 0, 0)),
        ],
        out_specs=[
            pl.BlockSpec((1, 3, BLK), lambda b, j: (b, 0, j)),
            pl.BlockSpec((1, 3, BLK), lambda b, j: (b, 0, j)),
        ],
        out_shape=[
            jax.ShapeDtypeStruct((B, 3, N), jnp.int32),
            jax.ShapeDtypeStruct((B, 3, N), jnp.float32),
        ],
    )(points_coords, centers_coords)


_GB = 128
_NG = _CHUNK // _GB


def _sc_gather_body(table_hbm, idx_hbm, out_hbm, idx_v, rows_v, sem):
    wid = lax.axis_index("s") * _NC + lax.axis_index("c")

    def chunk(t, _):
        base = wid * _PER_W + t * _CHUNK
        pltpu.sync_copy(idx_hbm.at[pl.ds(base // _GB, _NG)], idx_v)
        copies = [
            pltpu.async_copy(table_hbm.at[idx_v.at[j]],
                             rows_v.at[pl.ds(j * _GB, _GB)], sem)
            for j in range(_NG)
        ]
        for c in copies:
            c.wait()
        pltpu.sync_copy(rows_v, out_hbm.at[pl.ds(base, _CHUNK)])
        return ()

    lax.fori_loop(0, _NT, chunk, ())


@functools.cache
def _sc_gather():
    return pl.kernel(
        _sc_gather_body,
        out_type=jax.ShapeDtypeStruct((_FLAT, CIN), jnp.float32),
        mesh=plsc.VectorSubcoreMesh(core_axis_name="c", subcore_axis_name="s",
                                    num_cores=_NC, num_subcores=_NS),
        scratch_types=[
            pltpu.VMEM((_NG, _GB), jnp.int32),
            pltpu.VMEM((_CHUNK, CIN), jnp.float32),
            pltpu.SemaphoreType.DMA,
        ],
        compiler_params=pltpu.CompilerParams(use_tc_tiling_on_sc=False),
    )


def _mlp_body(g_ref, w_ref, w1_ref, b1_ref, w2_ref, b2_ref, o_ref):
    g = g_ref[0]
    w = w_ref[0]
    x = (g[0] * w[0][:, None] + g[1] * w[1][:, None] + g[2] * w[2][:, None])
    xt = jnp.swapaxes(x, 0, 1)
    h1 = jnp.maximum(
        lax.dot_general(w1_ref[...], xt, (((1,), (0,)), ((), ())),
                        preferred_element_type=jnp.float32) + b1_ref[...], 0.0)
    h2 = jnp.maximum(
        lax.dot_general(w2_ref[...], h1, (((1,), (0,)), ((), ())),
                        preferred_element_type=jnp.float32) + b2_ref[...], 0.0)
    o_ref[0] = h2


BLKC = 2048


def _mlp_call(gath, w, W1, b1, W2, b2):
    return pl.pallas_call(
        _mlp_body,
        grid=(B, N // BLKC),
        in_specs=[
            pl.BlockSpec((1, 3, BLKC, CIN), lambda b, j: (b, 0, j, 0)),
            pl.BlockSpec((1, 3, BLKC), lambda b, j: (b, 0, j)),
            pl.BlockSpec((C1, CIN), lambda b, j: (0, 0)),
            pl.BlockSpec((C1, 1), lambda b, j: (0, 0)),
            pl.BlockSpec((C2, C1), lambda b, j: (0, 0)),
            pl.BlockSpec((C2, 1), lambda b, j: (0, 0)),
        ],
        out_specs=pl.BlockSpec((1, C2, BLKC), lambda b, j: (b, 0, j)),
        out_shape=jax.ShapeDtypeStruct((B, C2, N), jnp.float32),
    )(gath, w, W1, b1.reshape(C1, 1), W2, b2.reshape(C2, 1))


def kernel(points_coords, centers_coords, centers_features, condition, W1, b1, W2, b2):
    idx, w = _knn_call(points_coords, centers_coords)
    table = jnp.swapaxes(centers_features, 1, 2).reshape(B * M, CIN)
    gath = _sc_gather()(table, idx.reshape(_FLAT // _GB, _GB))
    x = _mlp_call(gath.reshape(B, 3, N, CIN), w, W1, b1, W2, b2)
    return (x, points_coords, condition)


# speedup vs baseline: 1.5866x; 1.0694x over previous
import functools

import jax
import jax.numpy as jnp
import numpy as np
from jax import lax
from jax.experimental import pallas as pl
from jax.experimental.pallas import tpu as pltpu
from jax.experimental.pallas import tpu_sc as plsc

B, N, M, CIN, C1, C2 = 4, 16384, 1024, 32, 64, 64
BLK = 1024

_NC, _NS = 2, 16
_NW = _NC * _NS
_FLAT = 3 * B * N
_PER_W = _FLAT // _NW
_CHUNK = 2048
_NT = _PER_W // _CHUNK


def _knn_body(p_ref, c_ref, i_ref, w_ref):
    b = pl.program_id(0)
    p = p_ref[0]
    c = c_ref[0]
    pn2 = jnp.sum(p * p, axis=0)
    cm2 = jnp.sum(c * c, axis=0)
    cp = lax.dot_general(c, p, (((0,), (0,)), ((), ())),
                         preferred_element_type=jnp.float32)
    d2 = cm2[:, None] - 2.0 * cp + pn2[None, :]

    iota = lax.broadcasted_iota(jnp.int32, d2.shape, 0)
    d2i = lax.bitcast_convert_type(d2, jnp.int32)
    key = lax.bitcast_convert_type((d2i & ~(M - 1)) | iota, jnp.float32)

    inf = jnp.float32(np.inf)
    vals, idxs = [], []
    for k in range(3):
        v = jnp.min(key, axis=0)
        vi = lax.bitcast_convert_type(v, jnp.int32)
        idxs.append(vi & (M - 1))
        vals.append(lax.bitcast_convert_type(vi & ~(M - 1), jnp.float32))
        if k < 2:
            key = jnp.where(key == v[None, :], inf, key)

    w = [1.0 / (jnp.sqrt(jnp.maximum(v, 1e-10)) + 1e-8) for v in vals]
    wsum = w[0] + w[1] + w[2]
    i_ref[0] = jnp.concatenate([(i + b * M)[None] for i in idxs], axis=0)
    w_ref[0] = jnp.concatenate([(wi / wsum)[None] for wi in w], axis=0)


def _knn_call(points_coords, centers_coords):
    return pl.pallas_call(
        _knn_body,
        grid=(B, N // BLK),
        in_specs=[
            pl.BlockSpec((1, 3, BLK), lambda b, j: (b, 0, j)),
            pl.BlockSpec((1, 3, M), lambda b, j: (b, 0, 0)),
        ],
        out_specs=[
            pl.BlockSpec((1, 3, BLK), lambda b, j: (b, 0, j)),
            pl.BlockSpec((1, 3, BLK), lambda b, j: (b, 0, j)),
        ],
        out_shape=[
            jax.ShapeDtypeStruct((B, 3, N), jnp.int32),
            jax.ShapeDtypeStruct((B, 3, N), jnp.float32),
        ],
    )(points_coords, centers_coords)


_GB = 128
_NG = _CHUNK // _GB


def _sc_gather_body(table_hbm, idx_hbm, out_hbm, idx_v, rows_v, sem):
    wid = lax.axis_index("s") * _NC + lax.axis_index("c")

    def chunk(t, _):
        base = wid * _PER_W + t * _CHUNK
        pltpu.sync_copy(idx_hbm.at[pl.ds(base // _GB, _NG)], idx_v)
        copies = [
            pltpu.async_copy(table_hbm.at[idx_v.at[j]],
                             rows_v.at[pl.ds(j * _GB, _GB)], sem)
            for j in range(_NG)
        ]
        for c in copies:
            c.wait()
        pltpu.sync_copy(rows_v, out_hbm.at[pl.ds(base, _CHUNK)])
        return ()

    lax.fori_loop(0, _NT, chunk, ())


@functools.cache
def _sc_gather():
    return pl.kernel(
        _sc_gather_body,
        out_type=jax.ShapeDtypeStruct((_FLAT, CIN), jnp.float32),
        mesh=plsc.VectorSubcoreMesh(core_axis_name="c", subcore_axis_name="s",
                                    num_cores=_NC, num_subcores=_NS),
        scratch_types=[
            pltpu.VMEM((_NG, _GB), jnp.int32),
            pltpu.VMEM((_CHUNK, CIN), jnp.float32),
            pltpu.SemaphoreType.DMA,
        ],
        compiler_params=pltpu.CompilerParams(use_tc_tiling_on_sc=False),
    )


def _mlp_body(g_ref, w_ref, w1_ref, b1_ref, w2_ref, b2_ref, o_ref):
    g = g_ref[0]
    w = w_ref[0]
    x = (g[0] * w[0][:, None] + g[1] * w[1][:, None] + g[2] * w[2][:, None])
    xt = jnp.swapaxes(x, 0, 1)
    h1 = jnp.maximum(
        lax.dot_general(w1_ref[...], xt, (((1,), (0,)), ((), ())),
                        preferred_element_type=jnp.float32) + b1_ref[...], 0.0)
    h2 = jnp.maximum(
        lax.dot_general(w2_ref[...], h1, (((1,), (0,)), ((), ())),
                        preferred_element_type=jnp.float32) + b2_ref[...], 0.0)
    o_ref[0] = h2


BLKC = 2048


def _mlp_call(gath, w, W1, b1, W2, b2):
    return pl.pallas_call(
        _mlp_body,
        grid=(B, N // BLKC),
        in_specs=[
            pl.BlockSpec((1, 3, BLKC, CIN), lambda b, j: (b, 0, j, 0)),
            pl.BlockSpec((1, 3, BLKC), lambda b, j: (b, 0, j)),
            pl.BlockSpec((C1, CIN), lambda b, j: (0, 0)),
            pl.BlockSpec((C1, 1), lambda b, j: (0, 0)),
            pl.BlockSpec((C2, C1), lambda b, j: (0, 0)),
            pl.BlockSpec((C2, 1), lambda b, j: (0, 0)),
        ],
        out_specs=pl.BlockSpec((1, C2, BLKC), lambda b, j: (b, 0, j)),
        out_shape=jax.ShapeDtypeStruct((B, C2, N), jnp.float32),
    )(gath, w, W1, b1.reshape(C1, 1), W2, b2.reshape(C2, 1))


def kernel(points_coords, centers_coords, centers_features, condition, W1, b1, W2, b2):
    idx, w = _knn_call(points_coords, centers_coords)
    table = jnp.swapaxes(centers_features, 1, 2).reshape(B * M, CIN)
    gath = _sc_gather()(table, idx.reshape(_FLAT // _GB, _GB))
    x = _mlp_call(gath.reshape(B, 3, N, CIN), w, W1, b1, W2, b2)
    return (x, points_coords, condition)


# speedup vs baseline: 1.6133x; 1.0168x over previous
import functools

import jax
import jax.numpy as jnp
import numpy as np
from jax import lax
from jax.experimental import pallas as pl
from jax.experimental.pallas import tpu as pltpu
from jax.experimental.pallas import tpu_sc as plsc

B, N, M, CIN, C1, C2 = 4, 16384, 1024, 32, 64, 64
BLK = 2048

_NC, _NS = 2, 16
_NW = _NC * _NS
_FLAT = 3 * B * N
_PER_W = _FLAT // _NW
_CHUNK = 2048
_NT = _PER_W // _CHUNK


def _knn_body(p_ref, c_ref, i_ref, w_ref):
    b = pl.program_id(0)
    p = p_ref[0]
    c = c_ref[0]
    pn2 = jnp.sum(p * p, axis=0)
    cm2 = jnp.sum(c * c, axis=0)
    cp = lax.dot_general(c, p, (((0,), (0,)), ((), ())),
                         preferred_element_type=jnp.float32)
    d2 = cm2[:, None] - 2.0 * cp + pn2[None, :]

    iota = lax.broadcasted_iota(jnp.int32, d2.shape, 0)
    d2i = lax.bitcast_convert_type(d2, jnp.int32)
    key = lax.bitcast_convert_type((d2i & ~(M - 1)) | iota, jnp.float32)

    inf = jnp.float32(np.inf)
    vals, idxs = [], []
    for k in range(3):
        v = jnp.min(key, axis=0)
        vi = lax.bitcast_convert_type(v, jnp.int32)
        idxs.append(vi & (M - 1))
        vals.append(lax.bitcast_convert_type(vi & ~(M - 1), jnp.float32))
        if k < 2:
            key = jnp.where(key == v[None, :], inf, key)

    w = [1.0 / (jnp.sqrt(jnp.maximum(v, 1e-10)) + 1e-8) for v in vals]
    wsum = w[0] + w[1] + w[2]
    i_ref[0] = jnp.concatenate([(i + b * M)[None] for i in idxs], axis=0)
    w_ref[0] = jnp.concatenate([(wi / wsum)[None] for wi in w], axis=0)


def _knn_call(points_coords, centers_coords):
    return pl.pallas_call(
        _knn_body,
        grid=(B, N // BLK),
        in_specs=[
            pl.BlockSpec((1, 3, BLK), lambda b, j: (b, 0, j)),
            pl.BlockSpec((1, 3, M), lambda b, j: (b, 0, 0)),
        ],
        out_specs=[
            pl.BlockSpec((1, 3, BLK), lambda b, j: (b, 0, j)),
            pl.BlockSpec((1, 3, BLK), lambda b, j: (b, 0, j)),
        ],
        out_shape=[
            jax.ShapeDtypeStruct((B, 3, N), jnp.int32),
            jax.ShapeDtypeStruct((B, 3, N), jnp.float32),
        ],
    )(points_coords, centers_coords)


_GB = 128
_NG = _CHUNK // _GB


def _sc_gather_body(table_hbm, idx_hbm, out_hbm, idx_v, rows_v, sem):
    wid = lax.axis_index("s") * _NC + lax.axis_index("c")

    def chunk(t, _):
        base = wid * _PER_W + t * _CHUNK
        pltpu.sync_copy(idx_hbm.at[pl.ds(base // _GB, _NG)], idx_v)
        copies = [
            pltpu.async_copy(table_hbm.at[idx_v.at[j]],
                             rows_v.at[pl.ds(j * _GB, _GB)], sem)
            for j in range(_NG)
        ]
        for c in copies:
            c.wait()
        pltpu.sync_copy(rows_v, out_hbm.at[pl.ds(base, _CHUNK)])
        return ()

    lax.fori_loop(0, _NT, chunk, ())


@functools.cache
def _sc_gather():
    return pl.kernel(
        _sc_gather_body,
        out_type=jax.ShapeDtypeStruct((_FLAT, CIN), jnp.float32),
        mesh=plsc.VectorSubcoreMesh(core_axis_name="c", subcore_axis_name="s",
                                    num_cores=_NC, num_subcores=_NS),
        scratch_types=[
            pltpu.VMEM((_NG, _GB), jnp.int32),
            pltpu.VMEM((_CHUNK, CIN), jnp.float32),
            pltpu.SemaphoreType.DMA,
        ],
        compiler_params=pltpu.CompilerParams(use_tc_tiling_on_sc=False),
    )


def _mlp_body(g_ref, w_ref, w1_ref, b1_ref, w2_ref, b2_ref, o_ref):
    g = g_ref[0]
    w = w_ref[0]
    x = (g[0] * w[0][:, None] + g[1] * w[1][:, None] + g[2] * w[2][:, None])
    xt = jnp.swapaxes(x, 0, 1)
    h1 = jnp.maximum(
        lax.dot_general(w1_ref[...], xt, (((1,), (0,)), ((), ())),
                        preferred_element_type=jnp.float32) + b1_ref[...], 0.0)
    h2 = jnp.maximum(
        lax.dot_general(w2_ref[...], h1, (((1,), (0,)), ((), ())),
                        preferred_element_type=jnp.float32) + b2_ref[...], 0.0)
    o_ref[0] = h2


BLKC = 2048


def _mlp_call(gath, w, W1, b1, W2, b2):
    return pl.pallas_call(
        _mlp_body,
        grid=(B, N // BLKC),
        in_specs=[
            pl.BlockSpec((1, 3, BLKC, CIN), lambda b, j: (b, 0, j, 0)),
            pl.BlockSpec((1, 3, BLKC), lambda b, j: (b, 0, j)),
            pl.BlockSpec((C1, CIN), lambda b, j: (0, 0)),
            pl.BlockSpec((C1, 1), lambda b, j: (0, 0)),
            pl.BlockSpec((C2, C1), lambda b, j: (0, 0)),
            pl.BlockSpec((C2, 1), lambda b, j: (0, 0)),
        ],
        out_specs=pl.BlockSpec((1, C2, BLKC), lambda b, j: (b, 0, j)),
        out_shape=jax.ShapeDtypeStruct((B, C2, N), jnp.float32),
    )(gath, w, W1, b1.reshape(C1, 1), W2, b2.reshape(C2, 1))


def kernel(points_coords, centers_coords, centers_features, condition, W1, b1, W2, b2):
    idx, w = _knn_call(points_coords, centers_coords)
    table = jnp.swapaxes(centers_features, 1, 2).reshape(B * M, CIN)
    gath = _sc_gather()(table, idx.reshape(_FLAT // _GB, _GB))
    x = _mlp_call(gath.reshape(B, 3, N, CIN), w, W1, b1, W2, b2)
    return (x, points_coords, condition)


# speedup vs baseline: 1.7380x; 1.0773x over previous
import functools

import jax
import jax.numpy as jnp
import numpy as np
from jax import lax
from jax.experimental import pallas as pl
from jax.experimental.pallas import tpu as pltpu
from jax.experimental.pallas import tpu_sc as plsc

B, N, M, CIN, C1, C2 = 4, 16384, 1024, 32, 64, 64
BLK = 2048

_NC, _NS = 2, 16
_NW = _NC * _NS
_FLAT = 3 * B * N
_PER_W = _FLAT // _NW
_CHUNK = 2048
_NT = _PER_W // _CHUNK


def _knn_body(p_ref, c_ref, i_ref, w_ref):
    b = pl.program_id(0)
    p = p_ref[0]
    c = c_ref[0]
    pn2 = jnp.sum(p * p, axis=0)
    cm2 = jnp.sum(c * c, axis=0)
    cp = lax.dot_general(-2.0 * c, p, (((0,), (0,)), ((), ())),
                         preferred_element_type=jnp.float32)
    d2 = (cp + cm2[:, None]) + pn2[None, :]

    iota = lax.broadcasted_iota(jnp.int32, d2.shape, 0)
    d2i = lax.bitcast_convert_type(d2, jnp.int32)
    key = lax.bitcast_convert_type((d2i & ~(M - 1)) | iota, jnp.float32)

    inf = jnp.float32(np.inf)
    vals, idxs = [], []
    for k in range(3):
        v = jnp.min(key, axis=0)
        vi = lax.bitcast_convert_type(v, jnp.int32)
        idxs.append(vi & (M - 1))
        vals.append(lax.bitcast_convert_type(vi & ~(M - 1), jnp.float32))
        if k < 2:
            key = jnp.where(key == v[None, :], inf, key)

    w = [1.0 / (jnp.sqrt(jnp.maximum(v, 1e-10)) + 1e-8) for v in vals]
    wsum = w[0] + w[1] + w[2]
    i_ref[0] = jnp.concatenate([(i + b * M)[None] for i in idxs], axis=0)
    w_ref[0] = jnp.concatenate([(wi / wsum)[None] for wi in w], axis=0)


def _knn_call(points_coords, centers_coords):
    return pl.pallas_call(
        _knn_body,
        grid=(B, N // BLK),
        in_specs=[
            pl.BlockSpec((1, 3, BLK), lambda b, j: (b, 0, j)),
            pl.BlockSpec((1, 3, M), lambda b, j: (b, 0, 0)),
        ],
        out_specs=[
            pl.BlockSpec((1, 3, BLK), lambda b, j: (b, 0, j)),
            pl.BlockSpec((1, 3, BLK), lambda b, j: (b, 0, j)),
        ],
        out_shape=[
            jax.ShapeDtypeStruct((B, 3, N), jnp.int32),
            jax.ShapeDtypeStruct((B, 3, N), jnp.float32),
        ],
    )(points_coords, centers_coords)


_GB = 128
_NG = _CHUNK // _GB


def _sc_gather_body(table_hbm, idx_hbm, out_hbm, idx_v, rows_v, sem):
    wid = lax.axis_index("s") * _NC + lax.axis_index("c")

    def chunk(t, _):
        base = wid * _PER_W + t * _CHUNK
        pltpu.sync_copy(idx_hbm.at[pl.ds(base // _GB, _NG)], idx_v)
        copies = [
            pltpu.async_copy(table_hbm.at[idx_v.at[j]],
                             rows_v.at[pl.ds(j * _GB, _GB)], sem)
            for j in range(_NG)
        ]
        for c in copies:
            c.wait()
        pltpu.sync_copy(rows_v, out_hbm.at[pl.ds(base, _CHUNK)])
        return ()

    lax.fori_loop(0, _NT, chunk, ())


@functools.cache
def _sc_gather():
    return pl.kernel(
        _sc_gather_body,
        out_type=jax.ShapeDtypeStruct((_FLAT, CIN), jnp.float32),
        mesh=plsc.VectorSubcoreMesh(core_axis_name="c", subcore_axis_name="s",
                                    num_cores=_NC, num_subcores=_NS),
        scratch_types=[
            pltpu.VMEM((_NG, _GB), jnp.int32),
            pltpu.VMEM((_CHUNK, CIN), jnp.float32),
            pltpu.SemaphoreType.DMA,
        ],
        compiler_params=pltpu.CompilerParams(use_tc_tiling_on_sc=False),
    )


def _mlp_body(g_ref, w_ref, w1_ref, b1_ref, w2_ref, b2_ref, o_ref):
    g = g_ref[0]
    w = w_ref[0]
    x = (g[0] * w[0][:, None] + g[1] * w[1][:, None] + g[2] * w[2][:, None])
    xt = jnp.swapaxes(x, 0, 1)
    h1 = jnp.maximum(
        lax.dot_general(w1_ref[...], xt, (((1,), (0,)), ((), ())),
                        preferred_element_type=jnp.float32) + b1_ref[...], 0.0)
    h2 = jnp.maximum(
        lax.dot_general(w2_ref[...], h1, (((1,), (0,)), ((), ())),
                        preferred_element_type=jnp.float32) + b2_ref[...], 0.0)
    o_ref[0] = h2


BLKC = 4096


def _mlp_call(gath, w, W1, b1, W2, b2):
    return pl.pallas_call(
        _mlp_body,
        grid=(B, N // BLKC),
        in_specs=[
            pl.BlockSpec((1, 3, BLKC, CIN), lambda b, j: (b, 0, j, 0)),
            pl.BlockSpec((1, 3, BLKC), lambda b, j: (b, 0, j)),
            pl.BlockSpec((C1, CIN), lambda b, j: (0, 0)),
            pl.BlockSpec((C1, 1), lambda b, j: (0, 0)),
            pl.BlockSpec((C2, C1), lambda b, j: (0, 0)),
            pl.BlockSpec((C2, 1), lambda b, j: (0, 0)),
        ],
        out_specs=pl.BlockSpec((1, C2, BLKC), lambda b, j: (b, 0, j)),
        out_shape=jax.ShapeDtypeStruct((B, C2, N), jnp.float32),
    )(gath, w, W1, b1.reshape(C1, 1), W2, b2.reshape(C2, 1))


def kernel(points_coords, centers_coords, centers_features, condition, W1, b1, W2, b2):
    idx, w = _knn_call(points_coords, centers_coords)
    table = jnp.swapaxes(centers_features, 1, 2).reshape(B * M, CIN)
    gath = _sc_gather()(table, idx.reshape(_FLAT // _GB, _GB))
    x = _mlp_call(gath.reshape(B, 3, N, CIN), w, W1, b1, W2, b2)
    return (x, points_coords, condition)
